# baseline (speedup 1.0000x reference)
import functools

import jax
import jax.numpy as jnp
from jax import lax
from jax.experimental import pallas as pl
from jax.experimental.pallas import tpu as pltpu
from jax.experimental.pallas import tpu_sc as plsc

BATCH = 16384
EMB_DIM = 32
NUM_F = 4
LANES = 16
NUM_CORES = 2
NUM_SUBCORES = 16
NW = NUM_CORES * NUM_SUBCORES
BPW = BATCH // NW

ROWS = 100001
VT = 128
FULL_VT = ROWS // VT
TAIL = ROWS - FULL_VT * VT
TWORDS = ROWS * EMB_DIM


def _build_transpose():
    mesh = plsc.VectorSubcoreMesh(core_axis_name="c", subcore_axis_name="s")

    @functools.partial(
        pl.kernel,
        mesh=mesh,
        compiler_params=pltpu.CompilerParams(
            use_tc_tiling_on_sc=True, needs_layout_passes=False),
        out_type=jax.ShapeDtypeStruct((NUM_F * TWORDS,), jnp.float32),
        scratch_types=[
            pltpu.VMEM((EMB_DIM, VT), jnp.float32),
            pltpu.VMEM((VT * EMB_DIM,), jnp.float32),
            pltpu.SemaphoreType.DMA,
        ],
    )
    def transpose(tt0, tt1, tt2, tt3, tails, out, inbuf, slab, sem):
        wid = lax.axis_index("s") * NUM_CORES + lax.axis_index("c")
        lo = wid * FULL_VT // NW
        hi = (wid + 1) * FULL_VT // NW
        lane = lax.iota(jnp.int32, LANES)

        def per_table(f, tt):
            def body(vt, carry):
                pltpu.sync_copy(tt.at[:, pl.ds(vt * VT, VT)], inbuf)

                def tpose(vv, c2):
                    iv = jnp.full((LANES,), vv, jnp.int32)
                    for half in range(EMB_DIM // LANES):
                        g = plsc.load_gather(inbuf, [lane + half * LANES, iv])
                        slab[pl.ds(vv * EMB_DIM + half * LANES, LANES)] = g
                    return c2

                lax.fori_loop(0, VT, tpose, 0)
                pltpu.sync_copy(
                    slab,
                    out.at[pl.ds(f * TWORDS + vt * VT * EMB_DIM,
                                 VT * EMB_DIM)])
                return carry

            lax.fori_loop(lo, hi, body, 0)

        for f, tt in enumerate((tt0, tt1, tt2, tt3)):
            per_table(f, tt)

        @pl.when(wid == 0)
        def _():
            for f in range(NUM_F):
                pltpu.sync_copy(
                    tails.at[pl.ds(f * TAIL * EMB_DIM, TAIL * EMB_DIM)],
                    slab.at[pl.ds(0, TAIL * EMB_DIM)])
                pltpu.sync_copy(
                    slab.at[pl.ds(0, TAIL * EMB_DIM)],
                    out.at[pl.ds(f * TWORDS + FULL_VT * VT * EMB_DIM,
                                 TAIL * EMB_DIM)])

    return transpose


def _build_gather():
    mesh = plsc.VectorSubcoreMesh(core_axis_name="c", subcore_axis_name="s")

    @functools.partial(
        pl.kernel,
        mesh=mesh,
        compiler_params=pltpu.CompilerParams(use_tc_tiling_on_sc=False),
        out_type=jax.ShapeDtypeStruct((BATCH * EMB_DIM,), jnp.float32),
        scratch_types=[
            pltpu.VMEM((BPW,), jnp.int32),
            pltpu.VMEM((BPW,), jnp.int32),
            pltpu.VMEM((BPW,), jnp.int32),
            pltpu.VMEM((BPW,), jnp.int32),
            pltpu.VMEM((BPW, EMB_DIM), jnp.float32),
            pltpu.VMEM((BPW, EMB_DIM), jnp.float32),
            pltpu.VMEM((BPW, EMB_DIM), jnp.float32),
            pltpu.VMEM((BPW, EMB_DIM), jnp.float32),
            pltpu.VMEM((BPW * EMB_DIM,), jnp.float32),
            pltpu.SemaphoreType.DMA,
        ],
    )
    def gather(idxs_hbm, tbl, out_hbm,
               i0, i1, i2, i3, r0, r1, r2, r3, res, sem):
        wid = lax.axis_index("s") * NUM_CORES + lax.axis_index("c")
        base = wid * BPW
        pltpu.sync_copy(idxs_hbm.at[pl.ds(0 * BATCH + base, BPW)], i0)
        pltpu.sync_copy(idxs_hbm.at[pl.ds(1 * BATCH + base, BPW)], i1)
        pltpu.sync_copy(idxs_hbm.at[pl.ds(2 * BATCH + base, BPW)], i2)
        pltpu.sync_copy(idxs_hbm.at[pl.ds(3 * BATCH + base, BPW)], i3)
        for f, iref in enumerate((i1, i2, i3)):
            off = (f + 1) * ROWS
            for g in range(BPW // LANES):
                sl = pl.ds(g * LANES, LANES)
                iref[sl] = iref[sl] + off
        c0 = pltpu.async_copy(tbl.at[i0], r0, sem)
        c1 = pltpu.async_copy(tbl.at[i1], r1, sem)
        c2 = pltpu.async_copy(tbl.at[i2], r2, sem)
        c3 = pltpu.async_copy(tbl.at[i3], r3, sem)
        c0.wait()
        c1.wait()
        c2.wait()
        c3.wait()

        def add_body(j, carry):
            for k in range(EMB_DIM // LANES):
                sl = pl.ds(k * LANES, LANES)
                res[pl.ds(j * EMB_DIM + k * LANES, LANES)] = (
                    r0[j, sl] + r1[j, sl] + r2[j, sl] + r3[j, sl])
            return carry

        lax.fori_loop(0, BPW, add_body, 0)
        pltpu.sync_copy(res, out_hbm.at[pl.ds(base * EMB_DIM, BPW * EMB_DIM)])

    return gather


_TRANSPOSE = _build_transpose()
_GATHER = _build_gather()


def kernel(feature, label, table0, table1, table2, table3, node_table):
    del label, node_table
    tables = (table0, table1, table2, table3)
    tts = [t.T for t in tables]
    tails = jnp.concatenate(
        [t[FULL_VT * VT:].reshape(-1) for t in tables])
    flat = _TRANSPOSE(*tts, tails)
    big = flat.reshape(NUM_F * ROWS, EMB_DIM)
    idxs = feature.T.reshape(-1)
    out = _GATHER(idxs, big)
    return out.reshape(BATCH, EMB_DIM)


# speedup vs baseline: 1.5245x; 1.5245x over previous
import functools

import jax
import jax.numpy as jnp
from jax import lax
from jax.experimental import pallas as pl
from jax.experimental.pallas import tpu as pltpu
from jax.experimental.pallas import tpu_sc as plsc

BATCH = 16384
EMB_DIM = 32
NUM_F = 4
LANES = 16
NUM_CORES = 2
NUM_SUBCORES = 16
NW = NUM_CORES * NUM_SUBCORES
BPW = BATCH // NW

ROWS = 100001
VT = 128
FULL_VT = ROWS // VT
TAIL = ROWS - FULL_VT * VT
TWORDS = ROWS * EMB_DIM


def _build_transpose():
    mesh = plsc.VectorSubcoreMesh(core_axis_name="c", subcore_axis_name="s")

    NSLOT = (FULL_VT + NW - 1) // NW
    NPAIR = NSLOT // 2
    SLAB_W = VT * EMB_DIM

    @functools.partial(
        pl.kernel,
        mesh=mesh,
        compiler_params=pltpu.CompilerParams(
            use_tc_tiling_on_sc=True, needs_layout_passes=False),
        out_type=jax.ShapeDtypeStruct((NUM_F * TWORDS,), jnp.float32),
        scratch_types=[
            pltpu.VMEM((EMB_DIM, VT), jnp.float32),
            pltpu.VMEM((EMB_DIM, VT), jnp.float32),
            pltpu.VMEM((SLAB_W,), jnp.float32),
            pltpu.VMEM((SLAB_W,), jnp.float32),
            pltpu.SemaphoreType.DMA,
            pltpu.SemaphoreType.DMA,
            pltpu.SemaphoreType.DMA,
            pltpu.SemaphoreType.DMA,
        ],
    )
    def transpose(tt0, tt1, tt2, tt3, tails, out,
                  ina, inb, slaba, slabb, sia, sib, soa, sob):
        wid = lax.axis_index("s") * NUM_CORES + lax.axis_index("c")
        iot = lax.iota(jnp.int32, LANES) * EMB_DIM

        def fetch(tt, vt, buf, sem):
            pltpu.async_copy(tt.at[:, pl.ds(vt * VT, VT)], buf, sem)

        def wait_in(buf, sem):
            pltpu.make_async_copy(tt0.at[:, pl.ds(0, VT)], buf, sem).wait()

        def tpose(buf, slab):
            def cbody(c, carry):
                idx_c = iot + c
                for g in range(VT // LANES):
                    v = buf[c, pl.ds(g * LANES, LANES)]
                    plsc.store_scatter(slab, [idx_c + g * LANES * EMB_DIM], v)
                return carry
            lax.fori_loop(0, EMB_DIM, cbody, 0)

        def store(f, vt, slab, sem):
            pltpu.async_copy(
                slab, out.at[pl.ds(f * TWORDS + vt * SLAB_W, SLAB_W)], sem)

        def drain_out(slab, sem):
            pltpu.make_async_copy(out.at[pl.ds(0, SLAB_W)], slab, sem).wait()

        def per_table(f, tt):
            fetch(tt, wid, ina, sia)
            fetch(tt, NW + wid, inb, sib)

            def pair(p, carry):
                g0 = 2 * p
                vt0 = g0 * NW + wid
                wait_in(ina, sia)

                @pl.when(p > 0)
                def _():
                    drain_out(slaba, soa)
                tpose(ina, slaba)
                vt2 = (g0 + 2) * NW + wid

                @pl.when(vt2 < FULL_VT)
                def _():
                    fetch(tt, vt2, ina, sia)
                store(f, vt0, slaba, soa)

                vt1 = (g0 + 1) * NW + wid
                wait_in(inb, sib)

                @pl.when(p > 0)
                def _():
                    drain_out(slabb, sob)
                tpose(inb, slabb)
                vt3 = (g0 + 3) * NW + wid

                @pl.when(vt3 < FULL_VT)
                def _():
                    fetch(tt, vt3, inb, sib)
                store(f, vt1, slabb, sob)
                return carry

            lax.fori_loop(0, NPAIR, pair, 0)

            vtt = 2 * NPAIR * NW + wid

            @pl.when(vtt < FULL_VT)
            def _():
                wait_in(ina, sia)
                drain_out(slaba, soa)
                tpose(ina, slaba)
                store(f, vtt, slaba, soa)

            drain_out(slaba, soa)
            drain_out(slabb, sob)

        for f, tt in enumerate((tt0, tt1, tt2, tt3)):
            per_table(f, tt)

        @pl.when(wid == 0)
        def _():
            for f in range(NUM_F):
                pltpu.sync_copy(
                    tails.at[pl.ds(f * TAIL * EMB_DIM, TAIL * EMB_DIM)],
                    slaba.at[pl.ds(0, TAIL * EMB_DIM)])
                pltpu.sync_copy(
                    slaba.at[pl.ds(0, TAIL * EMB_DIM)],
                    out.at[pl.ds(f * TWORDS + FULL_VT * VT * EMB_DIM,
                                 TAIL * EMB_DIM)])

    return transpose


def _build_gather():
    mesh = plsc.VectorSubcoreMesh(core_axis_name="c", subcore_axis_name="s")

    @functools.partial(
        pl.kernel,
        mesh=mesh,
        compiler_params=pltpu.CompilerParams(use_tc_tiling_on_sc=False),
        out_type=jax.ShapeDtypeStruct((BATCH * EMB_DIM,), jnp.float32),
        scratch_types=[
            pltpu.VMEM((BPW,), jnp.int32),
            pltpu.VMEM((BPW,), jnp.int32),
            pltpu.VMEM((BPW,), jnp.int32),
            pltpu.VMEM((BPW,), jnp.int32),
            pltpu.VMEM((BPW, EMB_DIM), jnp.float32),
            pltpu.VMEM((BPW, EMB_DIM), jnp.float32),
            pltpu.VMEM((BPW, EMB_DIM), jnp.float32),
            pltpu.VMEM((BPW, EMB_DIM), jnp.float32),
            pltpu.VMEM((BPW * EMB_DIM,), jnp.float32),
            pltpu.SemaphoreType.DMA,
        ],
    )
    def gather(idxs_hbm, tbl, out_hbm,
               i0, i1, i2, i3, r0, r1, r2, r3, res, sem):
        wid = lax.axis_index("s") * NUM_CORES + lax.axis_index("c")
        base = wid * BPW
        pltpu.sync_copy(idxs_hbm.at[pl.ds(0 * BATCH + base, BPW)], i0)
        pltpu.sync_copy(idxs_hbm.at[pl.ds(1 * BATCH + base, BPW)], i1)
        pltpu.sync_copy(idxs_hbm.at[pl.ds(2 * BATCH + base, BPW)], i2)
        pltpu.sync_copy(idxs_hbm.at[pl.ds(3 * BATCH + base, BPW)], i3)
        for f, iref in enumerate((i1, i2, i3)):
            off = (f + 1) * ROWS
            for g in range(BPW // LANES):
                sl = pl.ds(g * LANES, LANES)
                iref[sl] = iref[sl] + off
        c0 = pltpu.async_copy(tbl.at[i0], r0, sem)
        c1 = pltpu.async_copy(tbl.at[i1], r1, sem)
        c2 = pltpu.async_copy(tbl.at[i2], r2, sem)
        c3 = pltpu.async_copy(tbl.at[i3], r3, sem)
        c0.wait()
        c1.wait()
        c2.wait()
        c3.wait()

        def add_body(j, carry):
            for k in range(EMB_DIM // LANES):
                sl = pl.ds(k * LANES, LANES)
                res[pl.ds(j * EMB_DIM + k * LANES, LANES)] = (
                    r0[j, sl] + r1[j, sl] + r2[j, sl] + r3[j, sl])
            return carry

        lax.fori_loop(0, BPW, add_body, 0)
        pltpu.sync_copy(res, out_hbm.at[pl.ds(base * EMB_DIM, BPW * EMB_DIM)])

    return gather


_TRANSPOSE = _build_transpose()
_GATHER = _build_gather()


def kernel(feature, label, table0, table1, table2, table3, node_table):
    del label, node_table
    tables = (table0, table1, table2, table3)
    tts = [t.T for t in tables]
    tails = jnp.concatenate(
        [t[FULL_VT * VT:].reshape(-1) for t in tables])
    flat = _TRANSPOSE(*tts, tails)
    big = flat.reshape(NUM_F * ROWS, EMB_DIM)
    idxs = feature.T.reshape(-1)
    out = _GATHER(idxs, big)
    return out.reshape(BATCH, EMB_DIM)


# speedup vs baseline: 2.8583x; 1.8749x over previous
import functools

import jax
import jax.numpy as jnp
from jax import lax
from jax.experimental import pallas as pl
from jax.experimental.pallas import tpu as pltpu
from jax.experimental.pallas import tpu_sc as plsc

BATCH = 16384
EMB_DIM = 32
NUM_F = 4
LANES = 16
NUM_CORES = 2
NUM_SUBCORES = 16
NW = NUM_CORES * NUM_SUBCORES
BPW = BATCH // NW

ROWS = 100001
VT = 128
FULL_VT = ROWS // VT
TAIL = ROWS - FULL_VT * VT
TWORDS = ROWS * EMB_DIM


def _build_transpose():
    mesh = plsc.VectorSubcoreMesh(core_axis_name="c", subcore_axis_name="s")

    NSLOT = (FULL_VT + NW - 1) // NW
    NPAIR = NSLOT // 2
    SLAB_W = VT * EMB_DIM

    @functools.partial(
        pl.kernel,
        mesh=mesh,
        compiler_params=pltpu.CompilerParams(
            use_tc_tiling_on_sc=True, needs_layout_passes=False),
        out_type=jax.ShapeDtypeStruct((NUM_F * TWORDS,), jnp.float32),
        scratch_types=[
            pltpu.VMEM((EMB_DIM, VT), jnp.float32),
            pltpu.VMEM((EMB_DIM, VT), jnp.float32),
            pltpu.VMEM((SLAB_W,), jnp.float32),
            pltpu.VMEM((SLAB_W,), jnp.float32),
            pltpu.SemaphoreType.DMA,
            pltpu.SemaphoreType.DMA,
            pltpu.SemaphoreType.DMA,
            pltpu.SemaphoreType.DMA,
        ],
    )
    def transpose(tt0, tt1, tt2, tt3, tails, out,
                  ina, inb, slaba, slabb, sia, sib, soa, sob):
        wid = lax.axis_index("s") * NUM_CORES + lax.axis_index("c")
        lane = lax.iota(jnp.int32, LANES)
        lane32 = lane * EMB_DIM
        rots = [(lane + r) & (LANES - 1) for r in range(LANES)]
        qs = [lane32 + rot for rot in rots]

        def fetch(tt, vt, buf, sem):
            pltpu.async_copy(tt.at[:, pl.ds(vt * VT, VT)], buf, sem)

        def wait_in(buf, sem):
            pltpu.make_async_copy(tt0.at[:, pl.ds(0, VT)], buf, sem).wait()

        def tpose(buf, slab):
            def vbody(v0, carry):
                vvec = lane + v0 * LANES
                for h in range(EMB_DIM // LANES):
                    s_s = v0 * LANES * EMB_DIM + h * LANES
                    for r in range(LANES):
                        cvec = rots[r] + h * LANES if h else rots[r]
                        g = plsc.load_gather(buf, [cvec, vvec])
                        plsc.store_scatter(slab, [qs[r] + s_s], g)
                return carry
            lax.fori_loop(0, VT // LANES, vbody, 0)

        def store(f, vt, slab, sem):
            pltpu.async_copy(
                slab, out.at[pl.ds(f * TWORDS + vt * SLAB_W, SLAB_W)], sem)

        def drain_out(slab, sem):
            pltpu.make_async_copy(out.at[pl.ds(0, SLAB_W)], slab, sem).wait()

        def per_table(f, tt):
            fetch(tt, wid, ina, sia)
            fetch(tt, NW + wid, inb, sib)

            def pair(p, carry):
                g0 = 2 * p
                vt0 = g0 * NW + wid
                wait_in(ina, sia)

                @pl.when(p > 0)
                def _():
                    drain_out(slaba, soa)
                tpose(ina, slaba)
                vt2 = (g0 + 2) * NW + wid

                @pl.when(vt2 < FULL_VT)
                def _():
                    fetch(tt, vt2, ina, sia)
                store(f, vt0, slaba, soa)

                vt1 = (g0 + 1) * NW + wid
                wait_in(inb, sib)

                @pl.when(p > 0)
                def _():
                    drain_out(slabb, sob)
                tpose(inb, slabb)
                vt3 = (g0 + 3) * NW + wid

                @pl.when(vt3 < FULL_VT)
                def _():
                    fetch(tt, vt3, inb, sib)
                store(f, vt1, slabb, sob)
                return carry

            lax.fori_loop(0, NPAIR, pair, 0)

            vtt = 2 * NPAIR * NW + wid

            @pl.when(vtt < FULL_VT)
            def _():
                wait_in(ina, sia)
                drain_out(slaba, soa)
                tpose(ina, slaba)
                store(f, vtt, slaba, soa)

            drain_out(slaba, soa)
            drain_out(slabb, sob)

        for f, tt in enumerate((tt0, tt1, tt2, tt3)):
            per_table(f, tt)

        @pl.when(wid == 0)
        def _():
            for f in range(NUM_F):
                pltpu.sync_copy(
                    tails.at[pl.ds(f * TAIL * EMB_DIM, TAIL * EMB_DIM)],
                    slaba.at[pl.ds(0, TAIL * EMB_DIM)])
                pltpu.sync_copy(
                    slaba.at[pl.ds(0, TAIL * EMB_DIM)],
                    out.at[pl.ds(f * TWORDS + FULL_VT * VT * EMB_DIM,
                                 TAIL * EMB_DIM)])

    return transpose


def _build_gather():
    mesh = plsc.VectorSubcoreMesh(core_axis_name="c", subcore_axis_name="s")

    @functools.partial(
        pl.kernel,
        mesh=mesh,
        compiler_params=pltpu.CompilerParams(use_tc_tiling_on_sc=False),
        out_type=jax.ShapeDtypeStruct((BATCH * EMB_DIM,), jnp.float32),
        scratch_types=[
            pltpu.VMEM((BPW,), jnp.int32),
            pltpu.VMEM((BPW,), jnp.int32),
            pltpu.VMEM((BPW,), jnp.int32),
            pltpu.VMEM((BPW,), jnp.int32),
            pltpu.VMEM((BPW, EMB_DIM), jnp.float32),
            pltpu.VMEM((BPW, EMB_DIM), jnp.float32),
            pltpu.VMEM((BPW, EMB_DIM), jnp.float32),
            pltpu.VMEM((BPW, EMB_DIM), jnp.float32),
            pltpu.VMEM((BPW * EMB_DIM,), jnp.float32),
            pltpu.SemaphoreType.DMA,
        ],
    )
    def gather(idxs_hbm, tbl, out_hbm,
               i0, i1, i2, i3, r0, r1, r2, r3, res, sem):
        wid = lax.axis_index("s") * NUM_CORES + lax.axis_index("c")
        base = wid * BPW
        pltpu.sync_copy(idxs_hbm.at[pl.ds(0 * BATCH + base, BPW)], i0)
        pltpu.sync_copy(idxs_hbm.at[pl.ds(1 * BATCH + base, BPW)], i1)
        pltpu.sync_copy(idxs_hbm.at[pl.ds(2 * BATCH + base, BPW)], i2)
        pltpu.sync_copy(idxs_hbm.at[pl.ds(3 * BATCH + base, BPW)], i3)
        for f, iref in enumerate((i1, i2, i3)):
            off = (f + 1) * ROWS
            for g in range(BPW // LANES):
                sl = pl.ds(g * LANES, LANES)
                iref[sl] = iref[sl] + off
        c0 = pltpu.async_copy(tbl.at[i0], r0, sem)
        c1 = pltpu.async_copy(tbl.at[i1], r1, sem)
        c2 = pltpu.async_copy(tbl.at[i2], r2, sem)
        c3 = pltpu.async_copy(tbl.at[i3], r3, sem)
        c0.wait()
        c1.wait()
        c2.wait()
        c3.wait()

        def add_body(j, carry):
            for k in range(EMB_DIM // LANES):
                sl = pl.ds(k * LANES, LANES)
                res[pl.ds(j * EMB_DIM + k * LANES, LANES)] = (
                    r0[j, sl] + r1[j, sl] + r2[j, sl] + r3[j, sl])
            return carry

        lax.fori_loop(0, BPW, add_body, 0)
        pltpu.sync_copy(res, out_hbm.at[pl.ds(base * EMB_DIM, BPW * EMB_DIM)])

    return gather


_TRANSPOSE = _build_transpose()
_GATHER = _build_gather()


def kernel(feature, label, table0, table1, table2, table3, node_table):
    del label, node_table
    tables = (table0, table1, table2, table3)
    tts = [t.T for t in tables]
    tails = jnp.concatenate(
        [t[FULL_VT * VT:].reshape(-1) for t in tables])
    flat = _TRANSPOSE(*tts, tails)
    big = flat.reshape(NUM_F * ROWS, EMB_DIM)
    idxs = feature.T.reshape(-1)
    out = _GATHER(idxs, big)
    return out.reshape(BATCH, EMB_DIM)
